# baseline (speedup 1.0000x reference)
import functools

import jax
import jax.numpy as jnp
from jax import lax
from jax.experimental import pallas as pl
from jax.experimental.pallas import tpu as pltpu
from jax.experimental.pallas import tpu_sc as plsc

N_NODES = 50000
DIM = 128
D_FEAT = 256
D_OUT = DIM + D_FEAT

CHUNK = 80
NUM_FULL = N_NODES // CHUNK
NUM_CHUNKS = NUM_FULL + (1 if N_NODES % CHUNK else 0)
TAIL_BASE = N_NODES - CHUNK
NBUF = 4


@functools.lru_cache(maxsize=None)
def _build():
    mesh = plsc.VectorSubcoreMesh(core_axis_name="c", subcore_axis_name="s")
    nc, ns = mesh.num_cores, mesh.num_subcores
    nw = nc * ns
    iters = -(-NUM_CHUNKS // nw)
    full_iters = (NUM_CHUNKS - nw) // nw + 1 if NUM_CHUNKS >= nw else 0

    @functools.partial(
        pl.kernel,
        out_type=jax.ShapeDtypeStruct((N_NODES, D_OUT), jnp.float32),
        mesh=mesh,
        scratch_types=[
            pltpu.VMEM((iters, CHUNK), jnp.int32),
            pltpu.VMEM((NBUF, CHUNK, DIM), jnp.float32),
            pltpu.VMEM((NBUF // 2, CHUNK, D_FEAT), jnp.float32),
            pltpu.VMEM_SHARED((16, NBUF // 2, CHUNK, D_FEAT), jnp.float32),
            pltpu.SemaphoreType.DMA,
        ] + [pltpu.SemaphoreType.DMA] * (4 * NBUF),
    )
    def body(x_hbm, lt_hbm, idx_hbm, out_hbm, idx_v, h_v, x_vt, x_vs, isem,
             *sems):
        gsem = sems[0:NBUF]
        wsem = sems[NBUF:2 * NBUF]
        xrsem = sems[2 * NBUF:3 * NBUF]
        xwsem = sems[3 * NBUF:4 * NBUF]
        sid = lax.axis_index("s")
        wid = sid * nc + lax.axis_index("c")

        def xbuf(p):
            return x_vt.at[p // 2] if p % 2 == 0 else x_vs.at[sid, p // 2]

        def chunk_base(i):
            c = jnp.minimum(wid + i * nw, NUM_CHUNKS - 1)
            b = jnp.where(c < NUM_FULL, c * CHUNK, TAIL_BASE)
            return pl.multiple_of(b, 8)

        bases = [chunk_base(i) for i in range(full_iters)]

        idx_cp = [
            pltpu.async_copy(idx_hbm.at[pl.ds(bases[i], CHUNK)], idx_v.at[i], isem)
            for i in range(full_iters)
        ]
        for cp in idx_cp:
            cp.wait()

        gathers = [None] * NBUF
        writes = [None] * NBUF
        xreads = [None] * NBUF
        xwrites = [None] * NBUF

        def finalize(j):
            q = j % NBUF
            gathers[q].wait()
            writes[q] = pltpu.async_copy(
                h_v.at[q],
                out_hbm.at[pl.ds(bases[j], CHUNK), pl.ds(0, DIM)],
                wsem[q])
            xreads[q].wait()
            xwrites[q] = pltpu.async_copy(
                xbuf(q),
                out_hbm.at[pl.ds(bases[j], CHUNK), pl.ds(DIM, D_FEAT)],
                xwsem[q])

        for i in range(full_iters):
            p = i % NBUF
            if writes[p] is not None:
                writes[p].wait()
            if xwrites[p] is not None:
                xwrites[p].wait()
            gathers[p] = pltpu.async_copy(lt_hbm.at[idx_v.at[i]], h_v.at[p], gsem[p])
            xreads[p] = pltpu.async_copy(
                x_hbm.at[pl.ds(bases[i], CHUNK)], xbuf(p), xrsem[p])
            if i >= 1:
                finalize(i - 1)
        finalize(full_iters - 1)
        for cp in writes + xwrites:
            if cp is not None:
                cp.wait()

        n_tail = NUM_CHUNKS - full_iters * nw
        if n_tail:
            @pl.when(wid < n_tail)
            def _():
                c = wid + full_iters * nw
                b = jnp.where(c < NUM_FULL, c * CHUNK, TAIL_BASE)
                b = pl.multiple_of(b, 8)
                pltpu.sync_copy(idx_hbm.at[pl.ds(b, CHUNK)], idx_v.at[0])
                pltpu.async_copy(lt_hbm.at[idx_v.at[0]], h_v.at[0], gsem[0]).wait()
                pltpu.sync_copy(h_v.at[0],
                                out_hbm.at[pl.ds(b, CHUNK), pl.ds(0, DIM)])
                pltpu.sync_copy(x_hbm.at[pl.ds(b, CHUNK)], xbuf(0))
                pltpu.sync_copy(xbuf(0),
                                out_hbm.at[pl.ds(b, CHUNK), pl.ds(DIM, D_FEAT)])

    return body


def kernel(x, lt, all_nodes):
    idx32 = all_nodes.astype(jnp.int32)
    return _build()(x, lt, idx32)


# speedup vs baseline: 1.0204x; 1.0204x over previous
import functools

import jax
import jax.numpy as jnp
from jax import lax
from jax.experimental import pallas as pl
from jax.experimental.pallas import tpu as pltpu
from jax.experimental.pallas import tpu_sc as plsc

N_NODES = 50000
DIM = 128
D_FEAT = 256
D_OUT = DIM + D_FEAT

CHUNK = 96
NUM_FULL = N_NODES // CHUNK
NUM_CHUNKS = NUM_FULL + (1 if N_NODES % CHUNK else 0)
TAIL_BASE = N_NODES - CHUNK
NBUF = 3


@functools.lru_cache(maxsize=None)
def _build():
    mesh = plsc.VectorSubcoreMesh(core_axis_name="c", subcore_axis_name="s")
    nc, ns = mesh.num_cores, mesh.num_subcores
    nw = nc * ns
    iters = -(-NUM_CHUNKS // nw)
    full_iters = (NUM_CHUNKS - nw) // nw + 1 if NUM_CHUNKS >= nw else 0

    @functools.partial(
        pl.kernel,
        out_type=jax.ShapeDtypeStruct((N_NODES, D_OUT), jnp.float32),
        mesh=mesh,
        scratch_types=[
            pltpu.VMEM((iters, CHUNK), jnp.int32),
            pltpu.VMEM((NBUF, CHUNK, DIM), jnp.float32),
            pltpu.VMEM_SHARED((16, NBUF, CHUNK, D_FEAT), jnp.float32),
            pltpu.SemaphoreType.DMA,
        ] + [pltpu.SemaphoreType.DMA] * (4 * NBUF),
    )
    def body(x_hbm, lt_hbm, idx_hbm, out_hbm, idx_v, h_v, x_v, isem, *sems):
        gsem = sems[0:NBUF]
        wsem = sems[NBUF:2 * NBUF]
        xrsem = sems[2 * NBUF:3 * NBUF]
        xwsem = sems[3 * NBUF:4 * NBUF]
        sid = lax.axis_index("s")
        wid = sid * nc + lax.axis_index("c")

        def chunk_base(i):
            c = jnp.minimum(wid + i * nw, NUM_CHUNKS - 1)
            b = jnp.where(c < NUM_FULL, c * CHUNK, TAIL_BASE)
            return pl.multiple_of(b, 8)

        bases = [chunk_base(i) for i in range(full_iters)]

        idx_cp = [
            pltpu.async_copy(idx_hbm.at[pl.ds(bases[i], CHUNK)], idx_v.at[i], isem)
            for i in range(full_iters)
        ]
        for cp in idx_cp:
            cp.wait()

        gathers = [None] * NBUF
        writes = [None] * NBUF
        xreads = [None] * NBUF
        xwrites = [None] * NBUF

        def finalize(j):
            q = j % NBUF
            gathers[q].wait()
            writes[q] = pltpu.async_copy(
                h_v.at[q],
                out_hbm.at[pl.ds(bases[j], CHUNK), pl.ds(0, DIM)],
                wsem[q])
            xreads[q].wait()
            xwrites[q] = pltpu.async_copy(
                x_v.at[sid, q],
                out_hbm.at[pl.ds(bases[j], CHUNK), pl.ds(DIM, D_FEAT)],
                xwsem[q])

        for i in range(full_iters):
            p = i % NBUF
            if writes[p] is not None:
                writes[p].wait()
            if xwrites[p] is not None:
                xwrites[p].wait()
            gathers[p] = pltpu.async_copy(lt_hbm.at[idx_v.at[i]], h_v.at[p], gsem[p])
            xreads[p] = pltpu.async_copy(
                x_hbm.at[pl.ds(bases[i], CHUNK)], x_v.at[sid, p], xrsem[p])
            if i >= 1:
                finalize(i - 1)
        finalize(full_iters - 1)
        for cp in writes + xwrites:
            if cp is not None:
                cp.wait()

        n_tail = NUM_CHUNKS - full_iters * nw
        if n_tail:
            @pl.when(wid < n_tail)
            def _():
                c = wid + full_iters * nw
                b = jnp.where(c < NUM_FULL, c * CHUNK, TAIL_BASE)
                b = pl.multiple_of(b, 8)
                pltpu.sync_copy(idx_hbm.at[pl.ds(b, CHUNK)], idx_v.at[0])
                pltpu.async_copy(lt_hbm.at[idx_v.at[0]], h_v.at[0], gsem[0]).wait()
                pltpu.sync_copy(h_v.at[0],
                                out_hbm.at[pl.ds(b, CHUNK), pl.ds(0, DIM)])
                pltpu.sync_copy(x_hbm.at[pl.ds(b, CHUNK)], x_v.at[sid, 0])
                pltpu.sync_copy(x_v.at[sid, 0],
                                out_hbm.at[pl.ds(b, CHUNK), pl.ds(DIM, D_FEAT)])

    return body


def kernel(x, lt, all_nodes):
    idx32 = all_nodes.astype(jnp.int32)
    return _build()(x, lt, idx32)


# speedup vs baseline: 1.0237x; 1.0032x over previous
import functools

import jax
import jax.numpy as jnp
from jax import lax
from jax.experimental import pallas as pl
from jax.experimental.pallas import tpu as pltpu
from jax.experimental.pallas import tpu_sc as plsc

N_NODES = 50000
DIM = 128
D_FEAT = 256
D_OUT = DIM + D_FEAT

CHUNK = 96
NUM_FULL = N_NODES // CHUNK
NUM_CHUNKS = NUM_FULL + (1 if N_NODES % CHUNK else 0)
TAIL_BASE = N_NODES - CHUNK
NBUF = 3


@functools.lru_cache(maxsize=None)
def _build():
    mesh = plsc.VectorSubcoreMesh(core_axis_name="c", subcore_axis_name="s")
    nc, ns = mesh.num_cores, mesh.num_subcores
    nw = nc * ns
    iters = -(-NUM_CHUNKS // nw)
    full_iters = (NUM_CHUNKS - nw) // nw + 1 if NUM_CHUNKS >= nw else 0

    @functools.partial(
        pl.kernel,
        out_type=jax.ShapeDtypeStruct((N_NODES, D_OUT), jnp.float32),
        mesh=mesh,
        scratch_types=[
            pltpu.VMEM((iters, CHUNK), jnp.int32),
            pltpu.VMEM((NBUF, CHUNK, DIM), jnp.float32),
            pltpu.VMEM_SHARED((ns, NBUF, CHUNK, D_FEAT), jnp.float32),
            pltpu.SemaphoreType.DMA,
        ] + [pltpu.SemaphoreType.DMA] * (4 * NBUF),
    )
    def body(x_hbm, lt_hbm, idx_hbm, out_hbm, idx_v, h_v, x_v, isem, *sems):
        gsem = sems[0:NBUF]
        wsem = sems[NBUF:2 * NBUF]
        xrsem = sems[2 * NBUF:3 * NBUF]
        xwsem = sems[3 * NBUF:4 * NBUF]
        sid = lax.axis_index("s")
        wid = sid * nc + lax.axis_index("c")

        def chunk_base(i):
            c = jnp.minimum(wid + i * nw, NUM_CHUNKS - 1)
            b = jnp.where(c < NUM_FULL, c * CHUNK, TAIL_BASE)
            return pl.multiple_of(b, 8)

        bases = [chunk_base(i) for i in range(full_iters)]

        idx_cp = [
            pltpu.async_copy(idx_hbm.at[pl.ds(bases[i], CHUNK)], idx_v.at[i], isem)
            for i in range(full_iters)
        ]
        for cp in idx_cp:
            cp.wait()

        gathers = [None] * NBUF
        writes = [None] * NBUF
        xreads = [None] * NBUF
        xwrites = [None] * NBUF

        def finalize(j):
            q = j % NBUF
            gathers[q].wait()
            writes[q] = pltpu.async_copy(
                h_v.at[q],
                out_hbm.at[pl.ds(bases[j], CHUNK), pl.ds(0, DIM)],
                wsem[q])
            xreads[q].wait()
            xwrites[q] = pltpu.async_copy(
                x_v.at[sid, q],
                out_hbm.at[pl.ds(bases[j], CHUNK), pl.ds(DIM, D_FEAT)],
                xwsem[q])

        for i in range(full_iters):
            p = i % NBUF
            if writes[p] is not None:
                writes[p].wait()
            if xwrites[p] is not None:
                xwrites[p].wait()
            gathers[p] = pltpu.async_copy(lt_hbm.at[idx_v.at[i]], h_v.at[p], gsem[p])
            xreads[p] = pltpu.async_copy(
                x_hbm.at[pl.ds(bases[i], CHUNK)], x_v.at[sid, p], xrsem[p])
            if i >= 1:
                finalize(i - 1)
        finalize(full_iters - 1)
        for cp in writes + xwrites:
            if cp is not None:
                cp.wait()

        n_tail = NUM_CHUNKS - full_iters * nw
        if n_tail:
            @pl.when(wid < n_tail)
            def _():
                c = wid + full_iters * nw
                b = jnp.where(c < NUM_FULL, c * CHUNK, TAIL_BASE)
                b = pl.multiple_of(b, 8)
                pltpu.sync_copy(idx_hbm.at[pl.ds(b, CHUNK)], idx_v.at[0])
                pltpu.async_copy(lt_hbm.at[idx_v.at[0]], h_v.at[0], gsem[0]).wait()
                pltpu.sync_copy(h_v.at[0],
                                out_hbm.at[pl.ds(b, CHUNK), pl.ds(0, DIM)])
                pltpu.sync_copy(x_hbm.at[pl.ds(b, CHUNK)], x_v.at[sid, 0])
                pltpu.sync_copy(x_v.at[sid, 0],
                                out_hbm.at[pl.ds(b, CHUNK), pl.ds(DIM, D_FEAT)])

    return body


def kernel(x, lt, all_nodes):
    idx32 = all_nodes.astype(jnp.int32)
    return _build()(x, lt, idx32)
